# baseline (speedup 1.0000x reference)
import functools

import jax
import jax.numpy as jnp
from jax import lax
from jax.experimental import pallas as pl
from jax.experimental.pallas import tpu as pltpu
from jax.experimental.pallas import tpu_sc as plsc

N_NODES = 10000
DIM = 256
QUART = 64
N_SEG = 10000
N_ENTRIES = 160000

NC = 2
NS = 16
L = 16

CHUNK = 128
NBUF = 4
SEG_PAD = 10112
SEG_CHUNKS = SEG_PAD // CHUNK
N_CHUNKS = 80
E_PER_SUB = N_CHUNKS * CHUNK
E_PAD = NS * E_PER_SUB

_mesh = plsc.VectorSubcoreMesh(core_axis_name="c", subcore_axis_name="s")


@jax.jit
def _sc_aggregate(table4, nid, sid, zeros_hbm):
    @functools.partial(
        pl.kernel,
        out_type=jax.ShapeDtypeStruct((N_SEG, DIM), jnp.float32),
        mesh=_mesh,
        compiler_params=pltpu.CompilerParams(use_tc_tiling_on_sc=False),
        scratch_types=[
            pltpu.VMEM((N_CHUNKS, CHUNK), jnp.int32),
            pltpu.VMEM((N_CHUNKS, CHUNK), jnp.int32),
            [pltpu.VMEM((CHUNK, QUART), jnp.float32)] * NBUF,
            pltpu.VMEM((CHUNK, QUART), jnp.float32),
            pltpu.VMEM((CHUNK, L), jnp.float32),
            pltpu.VMEM((CHUNK, L), jnp.float32),
            pltpu.VMEM_SHARED((SEG_PAD, QUART), jnp.float32),
            pltpu.VMEM_SHARED((SEG_PAD, L), jnp.float32),
            [pltpu.SemaphoreType.DMA] * NBUF,
            [pltpu.SemaphoreType.DMA] * NBUF,
            [pltpu.SemaphoreType.DMA] * NBUF,
        ],
    )
    def k(table_hbm, nid_hbm, sid_hbm, zeros_ref, out_hbm,
          nid_v, sid_v, rows, fin_v, ones_v, cnt_v, acc_sh, cnt_sh,
          gsem, ssem, csem):
        cid = lax.axis_index("c")
        sub = lax.axis_index("s")

        zero16 = jnp.zeros((L,), jnp.float32)
        one16 = jnp.ones((L,), jnp.float32)

        pltpu.sync_copy(nid_hbm.at[sub], nid_v)
        pltpu.sync_copy(sid_hbm.at[sub], sid_v)

        @pl.loop(0, CHUNK)
        def _(i):
            ones_v[i, pl.ds(0, L)] = one16
            cnt_v[i, pl.ds(0, L)] = zero16

        def adjust_indices(delta):
            @pl.loop(0, N_CHUNKS)
            def _(c):
                for j in range(CHUNK // L):
                    nid_v[c, pl.ds(j * L, L)] = (
                        nid_v[c, pl.ds(j * L, L)] + delta)

        def prime_gathers():
            for b in range(NBUF):
                pltpu.async_copy(table_hbm.at[nid_v.at[b]], rows[b], gsem[b])

        def accumulate(with_counts):
            @pl.loop(0, N_CHUNKS // NBUF)
            def _(z):
                for b in range(NBUF):
                    c = z * NBUF + b
                    pb = (b - 1) % NBUF
                    pltpu.make_async_copy(
                        table_hbm.at[nid_v.at[c]], rows[b], gsem[b]).wait()
                    pltpu.async_copy(rows[b], acc_sh.at[sid_v.at[c]],
                                     ssem[b], add=True)
                    if with_counts:
                        @pl.when(z > 0)
                        def _():
                            pltpu.make_async_copy(
                                ones_v, cnt_sh.at[sid_v.at[c]],
                                csem[b]).wait()

                        pltpu.async_copy(ones_v, cnt_sh.at[sid_v.at[c]],
                                         csem[b], add=True)

                    pc = jnp.maximum(c - 1, 0)
                    nxt = jnp.minimum(c - 1 + NBUF, N_CHUNKS - 1)

                    @pl.when(c > 0)
                    def _():
                        pltpu.make_async_copy(
                            rows[pb], acc_sh.at[sid_v.at[pc]],
                            ssem[pb]).wait()

                        @pl.when(c - 1 + NBUF < N_CHUNKS)
                        def _():
                            pltpu.async_copy(table_hbm.at[nid_v.at[nxt]],
                                             rows[pb], gsem[pb])

            lastb = (N_CHUNKS - 1) % NBUF
            pltpu.make_async_copy(
                rows[lastb], acc_sh.at[sid_v.at[N_CHUNKS - 1]],
                ssem[lastb]).wait()

            if with_counts:
                for b in range(NBUF):
                    pltpu.make_async_copy(
                        ones_v, cnt_sh.at[sid_v.at[0]], csem[b]).wait()

        def finalize(quart, rezero):
            @pl.loop(0, pl.cdiv(SEG_CHUNKS, NS))
            def _(f):
                t = f * NS + sub

                @pl.when(t < SEG_CHUNKS)
                def _():
                    base = t * CHUNK
                    pltpu.sync_copy(acc_sh.at[pl.ds(base, CHUNK)], fin_v)
                    pltpu.sync_copy(cnt_sh.at[pl.ds(base, CHUNK)], cnt_v)
                    if rezero:
                        pltpu.sync_copy(zeros_ref,
                                        acc_sh.at[pl.ds(base, CHUNK)])

                    @pl.loop(0, CHUNK)
                    def _(i):
                        cnt = cnt_v[i, pl.ds(0, L)]
                        inv = 1.0 / jnp.maximum(cnt, 1.0)
                        for j in range(QUART // L):
                            fin_v[i, pl.ds(j * L, L)] = (
                                fin_v[i, pl.ds(j * L, L)] * inv)

                    col = quart * QUART
                    TAIL = N_SEG - (SEG_CHUNKS - 1) * CHUNK

                    @pl.when(t < SEG_CHUNKS - 1)
                    def _():
                        pltpu.sync_copy(
                            fin_v,
                            out_hbm.at[pl.ds(base, CHUNK),
                                       pl.ds(col, QUART)])

                    @pl.when(t == SEG_CHUNKS - 1)
                    def _():
                        pltpu.sync_copy(
                            fin_v.at[pl.ds(0, TAIL)],
                            out_hbm.at[pl.ds(base, TAIL),
                                       pl.ds(col, QUART)])

        adjust_indices(2 * cid * N_NODES)
        prime_gathers()

        @pl.loop(0, pl.cdiv(SEG_CHUNKS, NS))
        def _(z):
            t = z * NS + sub

            @pl.when(t < SEG_CHUNKS)
            def _():
                pltpu.sync_copy(zeros_ref, acc_sh.at[pl.ds(t * CHUNK, CHUNK)])
                pltpu.sync_copy(cnt_v, cnt_sh.at[pl.ds(t * CHUNK, CHUNK)])

        plsc.subcore_barrier()
        accumulate(True)
        adjust_indices(N_NODES)
        prime_gathers()
        plsc.subcore_barrier()
        finalize(2 * cid, True)
        plsc.subcore_barrier()
        accumulate(False)
        plsc.subcore_barrier()
        finalize(2 * cid + 1, False)

    return k(table4, nid, sid, zeros_hbm)


def kernel(embedding_table, node_ids, segment_ids):
    nid = node_ids.astype(jnp.int32)
    sid = segment_ids.astype(jnp.int32)
    pad = E_PAD - N_ENTRIES
    nid = jnp.concatenate([nid, jnp.zeros((pad,), jnp.int32)])
    sid = jnp.concatenate([sid, jnp.full((pad,), N_SEG, jnp.int32)])
    nid = nid.reshape(NS, N_CHUNKS, CHUNK)
    sid = sid.reshape(NS, N_CHUNKS, CHUNK)
    table4 = jnp.concatenate(
        [embedding_table[:, q * QUART:(q + 1) * QUART] for q in range(4)],
        axis=0)
    zeros_hbm = jnp.zeros((CHUNK, QUART), jnp.float32)
    return _sc_aggregate(table4, nid, sid, zeros_hbm)


# speedup vs baseline: 1.1404x; 1.1404x over previous
import functools

import jax
import jax.numpy as jnp
from jax import lax
from jax.experimental import pallas as pl
from jax.experimental.pallas import tpu as pltpu
from jax.experimental.pallas import tpu_sc as plsc

N_NODES = 10000
DIM = 256
QUART = 64
N_SEG = 10000
N_ENTRIES = 160000

NC = 2
NS = 16
L = 16

CHUNK = 128
NBUF = 4
SEG_PAD = 10112
SEG_CHUNKS = SEG_PAD // CHUNK
N_CHUNKS = 80
E_PER_SUB = N_CHUNKS * CHUNK
E_PAD = NS * E_PER_SUB

_mesh = plsc.VectorSubcoreMesh(core_axis_name="c", subcore_axis_name="s")


@jax.jit
def _sc_aggregate(table4, nid, sid, zeros_hbm):
    @functools.partial(
        pl.kernel,
        out_type=jax.ShapeDtypeStruct((N_SEG, DIM), jnp.float32),
        mesh=_mesh,
        compiler_params=pltpu.CompilerParams(use_tc_tiling_on_sc=False),
        scratch_types=[
            pltpu.VMEM((N_CHUNKS, CHUNK), jnp.int32),
            pltpu.VMEM((N_CHUNKS, CHUNK), jnp.int32),
            [pltpu.VMEM((CHUNK, QUART), jnp.float32)] * NBUF,
            pltpu.VMEM((CHUNK, QUART), jnp.float32),
            pltpu.VMEM((CHUNK, L), jnp.float32),
            pltpu.VMEM((CHUNK, L), jnp.float32),
            pltpu.VMEM_SHARED((SEG_PAD, QUART), jnp.float32),
            pltpu.VMEM_SHARED((SEG_PAD, L), jnp.float32),
            [pltpu.SemaphoreType.DMA] * NBUF,
            [pltpu.SemaphoreType.DMA] * NBUF,
            [pltpu.SemaphoreType.DMA] * NBUF,
        ],
    )
    def k(table_hbm, nid_hbm, sid_hbm, zeros_ref, out_hbm,
          nid_v, sid_v, rows, fin_v, ones_v, cnt_v, acc_sh, cnt_sh,
          gsem, ssem, csem):
        cid = lax.axis_index("c")
        sub = lax.axis_index("s")

        zero16 = jnp.zeros((L,), jnp.float32)
        one16 = jnp.ones((L,), jnp.float32)

        pltpu.sync_copy(nid_hbm.at[sub], nid_v)
        pltpu.sync_copy(sid_hbm.at[sub], sid_v)

        @pl.loop(0, CHUNK)
        def _(i):
            ones_v[i, pl.ds(0, L)] = one16
            cnt_v[i, pl.ds(0, L)] = zero16

        def adjust_indices(delta):
            @pl.loop(0, N_CHUNKS)
            def _(c):
                for j in range(CHUNK // L):
                    nid_v[c, pl.ds(j * L, L)] = (
                        nid_v[c, pl.ds(j * L, L)] + delta)

        def prime_gathers():
            for b in range(NBUF):
                pltpu.async_copy(table_hbm.at[nid_v.at[b]], rows[b], gsem[b])

        def accumulate(with_counts):
            @pl.loop(0, N_CHUNKS // NBUF)
            def _(z):
                for b in range(NBUF):
                    c = z * NBUF + b
                    pb = (b - 1) % NBUF
                    pltpu.make_async_copy(
                        table_hbm.at[nid_v.at[c]], rows[b], gsem[b]).wait()
                    pltpu.async_copy(rows[b], acc_sh.at[sid_v.at[c]],
                                     ssem[b], add=True)
                    if with_counts:
                        @pl.when(z > 0)
                        def _():
                            pltpu.make_async_copy(
                                ones_v, cnt_sh.at[sid_v.at[c]],
                                csem[b]).wait()

                        pltpu.async_copy(ones_v, cnt_sh.at[sid_v.at[c]],
                                         csem[b], add=True)

                    pc = jnp.maximum(c - 1, 0)
                    nxt = jnp.minimum(c - 1 + NBUF, N_CHUNKS - 1)

                    @pl.when(c > 0)
                    def _():
                        pltpu.make_async_copy(
                            rows[pb], acc_sh.at[sid_v.at[pc]],
                            ssem[pb]).wait()

                        @pl.when(c - 1 + NBUF < N_CHUNKS)
                        def _():
                            pltpu.async_copy(table_hbm.at[nid_v.at[nxt]],
                                             rows[pb], gsem[pb])

            lastb = (N_CHUNKS - 1) % NBUF
            pltpu.make_async_copy(
                rows[lastb], acc_sh.at[sid_v.at[N_CHUNKS - 1]],
                ssem[lastb]).wait()

            if with_counts:
                for b in range(NBUF):
                    pltpu.make_async_copy(
                        ones_v, cnt_sh.at[sid_v.at[0]], csem[b]).wait()

        def finalize(quart, rezero):
            @pl.loop(0, pl.cdiv(SEG_CHUNKS, NS))
            def _(f):
                t = f * NS + sub

                @pl.when(t < SEG_CHUNKS)
                def _():
                    base = t * CHUNK
                    pltpu.sync_copy(acc_sh.at[pl.ds(base, CHUNK)], fin_v)
                    pltpu.sync_copy(cnt_sh.at[pl.ds(base, CHUNK)], cnt_v)
                    if rezero:
                        pltpu.sync_copy(zeros_ref,
                                        acc_sh.at[pl.ds(base, CHUNK)])

                    @pl.loop(0, CHUNK)
                    def _(i):
                        cnt = cnt_v[i, pl.ds(0, L)]
                        inv = 1.0 / jnp.maximum(cnt, 1.0)
                        for j in range(QUART // L):
                            fin_v[i, pl.ds(j * L, L)] = (
                                fin_v[i, pl.ds(j * L, L)] * inv)

                    col = quart * QUART
                    TAIL = N_SEG - (SEG_CHUNKS - 1) * CHUNK

                    @pl.when(t < SEG_CHUNKS - 1)
                    def _():
                        pltpu.sync_copy(
                            fin_v,
                            out_hbm.at[pl.ds(base, CHUNK),
                                       pl.ds(col, QUART)])

                    @pl.when(t == SEG_CHUNKS - 1)
                    def _():
                        pltpu.sync_copy(
                            fin_v.at[pl.ds(0, TAIL)],
                            out_hbm.at[pl.ds(base, TAIL),
                                       pl.ds(col, QUART)])

        adjust_indices(2 * cid)
        prime_gathers()

        @pl.loop(0, pl.cdiv(SEG_CHUNKS, NS))
        def _(z):
            t = z * NS + sub

            @pl.when(t < SEG_CHUNKS)
            def _():
                pltpu.sync_copy(zeros_ref, acc_sh.at[pl.ds(t * CHUNK, CHUNK)])
                pltpu.sync_copy(cnt_v, cnt_sh.at[pl.ds(t * CHUNK, CHUNK)])

        plsc.subcore_barrier()
        accumulate(True)
        adjust_indices(1)
        prime_gathers()
        plsc.subcore_barrier()
        finalize(2 * cid, True)
        plsc.subcore_barrier()
        accumulate(False)
        plsc.subcore_barrier()
        finalize(2 * cid + 1, False)

    return k(table4, nid, sid, zeros_hbm)


def kernel(embedding_table, node_ids, segment_ids):
    table4 = embedding_table.reshape(4 * N_NODES, QUART)
    nid = node_ids.astype(jnp.int32) * 4
    sid = segment_ids.astype(jnp.int32)
    pad = E_PAD - N_ENTRIES
    nid = jnp.concatenate([nid, jnp.zeros((pad,), jnp.int32)])
    sid = jnp.concatenate([sid, jnp.full((pad,), N_SEG, jnp.int32)])
    nid = nid.reshape(NS, N_CHUNKS, CHUNK)
    sid = sid.reshape(NS, N_CHUNKS, CHUNK)
    zeros_hbm = jnp.zeros((CHUNK, QUART), jnp.float32)
    return _sc_aggregate(table4, nid, sid, zeros_hbm)
